# baseline (speedup 1.0000x reference)
import jax
import jax.numpy as jnp
from jax.experimental import pallas as pl
from jax.experimental.pallas import tpu as pltpu

_N_E = 1792
_E_DIM = 256
_K = 256
_BETA = 0.25
_BLOCK = 4096
_BATCH_ROWS = 1024
_HIGH = jax.lax.Precision.HIGHEST


def _vq_block_kernel(pos_ref, z_ref, w_ref,
                     zqt_ref, loss_ref, perp_ref, enc_ref, idx_ref,
                     counts_ref, lacc_ref, wt_ref, w2_ref):
    i = pl.program_id(0)
    nsteps = pl.num_programs(0)

    w = w_ref[...]

    @pl.when(i == 0)
    def _init():
        counts_ref[...] = jnp.zeros_like(counts_ref)
        lacc_ref[0, 0] = 0.0
        wtv = w.T
        wt_ref[...] = wtv
        w2_ref[...] = jnp.sum(wtv * wtv, axis=0, keepdims=True)

    z = z_ref[0]
    w2 = w2_ref[...]

    s = jax.lax.dot_general(z, w, (((1,), (1,)), ((), ())),
                            preferred_element_type=jnp.float32,
                            precision=jax.lax.Precision.DEFAULT)
    z2 = jnp.sum(z * z, axis=1, keepdims=True)
    d = (z2 + w2) - 2.0 * s

    dmin = jnp.min(d, axis=1, keepdims=True)
    lane = jax.lax.broadcasted_iota(jnp.int32, (_BLOCK, _K), 1)
    idx = jnp.min(jnp.where(d == dmin, lane, _K), axis=1, keepdims=True)

    enc = (lane == idx).astype(jnp.float32)
    enc_ref[...] = enc
    counts_ref[...] += jnp.sum(enc, axis=0, keepdims=True)

    lacc_ref[0, 0] += jnp.sum(dmin)

    wt = wt_ref[...]
    lane_row = jax.lax.broadcasted_iota(
        jnp.int32, (1, _K), 1).astype(jnp.float32)
    for b in range(_BLOCK // _BATCH_ROWS):
        enc_b = enc[b * _BATCH_ROWS:(b + 1) * _BATCH_ROWS]
        zqt_ref[b] = jax.lax.dot_general(wt, enc_b, (((1,), (1,)), ((), ())),
                                         preferred_element_type=jnp.float32,
                                         precision=jax.lax.Precision.DEFAULT)
        idxr = jax.lax.dot_general(lane_row, enc_b, (((1,), (1,)), ((), ())),
                                   preferred_element_type=jnp.float32,
                                   precision=jax.lax.Precision.DEFAULT)
        idx_ref[b] = idxr.astype(jnp.int32)

    @pl.when(i == nsteps - 1)
    def _finish():
        total = jnp.float32(nsteps * _BLOCK)
        loss = (1.0 + _BETA) * lacc_ref[0, 0] / (total * _E_DIM)
        loss_ref[...] = jnp.reshape(loss, (1, 1))
        e_mean = counts_ref[...] * (1.0 / total)
        ent = jnp.sum(e_mean * jnp.log(e_mean + 1e-10))
        perp_ref[...] = jnp.reshape(jnp.exp(-ent), (1, 1))


def kernel(z, one_hot, W):
    n_rows = z.shape[0] * z.shape[1]
    nsteps = n_rows // _BLOCK
    zr = z.reshape(nsteps, _BLOCK, _E_DIM)
    pos = jnp.argmax(one_hot).astype(jnp.int32)[None]

    grid_spec = pltpu.PrefetchScalarGridSpec(
        num_scalar_prefetch=1,
        grid=(nsteps,),
        in_specs=[
            pl.BlockSpec((1, _BLOCK, _E_DIM), lambda i, pos: (i, 0, 0)),
            pl.BlockSpec((_K, _E_DIM), lambda i, pos: (pos[0], 0)),
        ],
        out_specs=[
            pl.BlockSpec((_BLOCK // _BATCH_ROWS, _E_DIM, _BATCH_ROWS),
                         lambda i, pos: (i, 0, 0)),
            pl.BlockSpec((1, 1), lambda i, pos: (0, 0)),
            pl.BlockSpec((1, 1), lambda i, pos: (0, 0)),
            pl.BlockSpec((_BLOCK, _K), lambda i, pos: (i, 0)),
            pl.BlockSpec((_BLOCK // _BATCH_ROWS, 1, _BATCH_ROWS),
                         lambda i, pos: (i, 0, 0)),
        ],
        scratch_shapes=[
            pltpu.VMEM((1, _K), jnp.float32),
            pltpu.SMEM((1, 1), jnp.float32),
            pltpu.VMEM((_E_DIM, _K), jnp.float32),
            pltpu.VMEM((1, _K), jnp.float32),
        ],
    )
    zqt, loss, perp, enc, idx = pl.pallas_call(
        _vq_block_kernel,
        grid_spec=grid_spec,
        out_shape=[
            jax.ShapeDtypeStruct((z.shape[0], _E_DIM, _BATCH_ROWS), z.dtype),
            jax.ShapeDtypeStruct((1, 1), jnp.float32),
            jax.ShapeDtypeStruct((1, 1), jnp.float32),
            jax.ShapeDtypeStruct((n_rows, _K), z.dtype),
            jax.ShapeDtypeStruct((z.shape[0], 1, _BATCH_ROWS), jnp.int32),
        ],
        compiler_params=pltpu.CompilerParams(
            dimension_semantics=("arbitrary",)),
    )(pos, zr, W)
    return (zqt, loss.reshape(()),
            (perp.reshape(()), enc, idx.reshape(n_rows, 1)))


# speedup vs baseline: 1.0262x; 1.0262x over previous
import jax
import jax.numpy as jnp
from jax.experimental import pallas as pl
from jax.experimental.pallas import tpu as pltpu

_N_E = 1792
_E_DIM = 256
_K = 256
_BETA = 0.25
_BLOCK = 8192
_BATCH_ROWS = 1024
_HIGH = jax.lax.Precision.HIGHEST


def _vq_block_kernel(pos_ref, z_ref, w_ref,
                     zqt_ref, loss_ref, perp_ref, enc_ref, idx_ref,
                     counts_ref, lacc_ref, wt_ref, w2_ref):
    i = pl.program_id(0)
    nsteps = pl.num_programs(0)

    w = w_ref[...]

    @pl.when(i == 0)
    def _init():
        counts_ref[...] = jnp.zeros_like(counts_ref)
        lacc_ref[0, 0] = 0.0
        wtv = w.T
        wt_ref[...] = wtv
        w2_ref[...] = jnp.sum(wtv * wtv, axis=0, keepdims=True)

    z = z_ref[0]
    w2 = w2_ref[...]

    s = jax.lax.dot_general(z, w, (((1,), (1,)), ((), ())),
                            preferred_element_type=jnp.float32,
                            precision=jax.lax.Precision.DEFAULT)
    z2 = jnp.sum(z * z, axis=1, keepdims=True)
    d = (z2 + w2) - 2.0 * s

    dmin = jnp.min(d, axis=1, keepdims=True)
    lane = jax.lax.broadcasted_iota(jnp.int32, (_BLOCK, _K), 1)
    idx = jnp.min(jnp.where(d == dmin, lane, _K), axis=1, keepdims=True)

    enc = (lane == idx).astype(jnp.float32)
    enc_ref[...] = enc
    counts_ref[...] += jnp.sum(enc, axis=0, keepdims=True)

    lacc_ref[0, 0] += jnp.sum(dmin)

    wt = wt_ref[...]
    lane_row = jax.lax.broadcasted_iota(
        jnp.int32, (1, _K), 1).astype(jnp.float32)
    for b in range(_BLOCK // _BATCH_ROWS):
        enc_b = enc[b * _BATCH_ROWS:(b + 1) * _BATCH_ROWS]
        zqt_ref[b] = jax.lax.dot_general(wt, enc_b, (((1,), (1,)), ((), ())),
                                         preferred_element_type=jnp.float32,
                                         precision=jax.lax.Precision.DEFAULT)
        idxr = jax.lax.dot_general(lane_row, enc_b, (((1,), (1,)), ((), ())),
                                   preferred_element_type=jnp.float32,
                                   precision=jax.lax.Precision.DEFAULT)
        idx_ref[b] = idxr.astype(jnp.int32)

    @pl.when(i == nsteps - 1)
    def _finish():
        total = jnp.float32(nsteps * _BLOCK)
        loss = (1.0 + _BETA) * lacc_ref[0, 0] / (total * _E_DIM)
        loss_ref[...] = jnp.reshape(loss, (1, 1))
        e_mean = counts_ref[...] * (1.0 / total)
        ent = jnp.sum(e_mean * jnp.log(e_mean + 1e-10))
        perp_ref[...] = jnp.reshape(jnp.exp(-ent), (1, 1))


def kernel(z, one_hot, W):
    n_rows = z.shape[0] * z.shape[1]
    nsteps = n_rows // _BLOCK
    zr = z.reshape(nsteps, _BLOCK, _E_DIM)
    pos = jnp.argmax(one_hot).astype(jnp.int32)[None]

    grid_spec = pltpu.PrefetchScalarGridSpec(
        num_scalar_prefetch=1,
        grid=(nsteps,),
        in_specs=[
            pl.BlockSpec((1, _BLOCK, _E_DIM), lambda i, pos: (i, 0, 0)),
            pl.BlockSpec((_K, _E_DIM), lambda i, pos: (pos[0], 0)),
        ],
        out_specs=[
            pl.BlockSpec((_BLOCK // _BATCH_ROWS, _E_DIM, _BATCH_ROWS),
                         lambda i, pos: (i, 0, 0)),
            pl.BlockSpec((1, 1), lambda i, pos: (0, 0)),
            pl.BlockSpec((1, 1), lambda i, pos: (0, 0)),
            pl.BlockSpec((_BLOCK, _K), lambda i, pos: (i, 0)),
            pl.BlockSpec((_BLOCK // _BATCH_ROWS, 1, _BATCH_ROWS),
                         lambda i, pos: (i, 0, 0)),
        ],
        scratch_shapes=[
            pltpu.VMEM((1, _K), jnp.float32),
            pltpu.SMEM((1, 1), jnp.float32),
            pltpu.VMEM((_E_DIM, _K), jnp.float32),
            pltpu.VMEM((1, _K), jnp.float32),
        ],
    )
    zqt, loss, perp, enc, idx = pl.pallas_call(
        _vq_block_kernel,
        grid_spec=grid_spec,
        out_shape=[
            jax.ShapeDtypeStruct((z.shape[0], _E_DIM, _BATCH_ROWS), z.dtype),
            jax.ShapeDtypeStruct((1, 1), jnp.float32),
            jax.ShapeDtypeStruct((1, 1), jnp.float32),
            jax.ShapeDtypeStruct((n_rows, _K), z.dtype),
            jax.ShapeDtypeStruct((z.shape[0], 1, _BATCH_ROWS), jnp.int32),
        ],
        compiler_params=pltpu.CompilerParams(
            dimension_semantics=("arbitrary",)),
    )(pos, zr, W)
    return (zqt, loss.reshape(()),
            (perp.reshape(()), enc, idx.reshape(n_rows, 1)))
